# baseline (device time: 97404 ns/iter reference)
import jax
import jax.numpy as jnp
from jax import lax
from jax.experimental import pallas as pl
from jax.experimental.pallas import tpu as pltpu

N_DEV = 4
S_PER = 512
SEQ = N_DEV * S_PER
D = 1024
H = 8
DH = 128
SCALE = 0.08838834764831843
BF16 = jnp.bfloat16
F32 = jnp.float32


def _body(x_ref, wq_ref, wo_ref, wk_ref, wv_ref, out_ref,
          xc, qb, kb, vb, attnbuf, pacc, pacc2, rsbuf,
          ag_send, ag_recv, rs_send, rs_recv):
    i = lax.axis_index("i")
    left = (i + N_DEV - 1) % N_DEV
    right = (i + 1) % N_DEV

    barrier = pltpu.get_barrier_semaphore()
    for nbr in (left, right):
        pl.semaphore_signal(barrier, inc=1, device_id=(nbr,),
                            device_id_type=pl.DeviceIdType.MESH)
    pl.semaphore_wait(barrier, 2)

    xc[pl.ds(i * S_PER, S_PER), :] = x_ref[...].astype(BF16)
    wq = wq_ref[...].astype(BF16)
    wk = wk_ref[...].astype(BF16)
    wv = wv_ref[...].astype(BF16)

    def project(c):
        rows = pl.ds(c * S_PER, S_PER)
        xv = xc[rows, :]
        qb[rows, :] = jnp.dot(xv, wq, preferred_element_type=F32).astype(BF16)
        kb[rows, :] = jnp.dot(xv, wk, preferred_element_type=F32).astype(BF16)
        vb[rows, :] = jnp.dot(xv, wv, preferred_element_type=F32).astype(BF16)

    def chunk_copy(c, sem_idx, target):
        return pltpu.make_async_remote_copy(
            src_ref=xc.at[pl.ds(c * S_PER, S_PER)],
            dst_ref=xc.at[pl.ds(c * S_PER, S_PER)],
            send_sem=ag_send.at[sem_idx],
            recv_sem=ag_recv.at[sem_idx],
            device_id=(target,),
            device_id_type=pl.DeviceIdType.MESH,
        )

    rdma_r = chunk_copy(i, 0, right)
    rdma_l = chunk_copy(i, 1, left)
    rdma_r.start()
    rdma_l.start()
    project(i)
    rdma_r.wait_recv()
    rdma_f = chunk_copy((i + N_DEV - 1) % N_DEV, 2, right)
    rdma_f.start()
    rdma_l.wait_recv()
    project((i + N_DEV - 1) % N_DEV)
    project((i + 1) % N_DEV)
    rdma_f.wait_recv()
    project((i + 2) % N_DEV)
    rdma_r.wait_send()
    rdma_l.wait_send()
    rdma_f.wait_send()

    wo = wo_ref[...].astype(BF16)

    def pchunk(c):
        rows = pl.ds(c * S_PER, S_PER)

        for h in range(H):
            cols = slice(h * DH, (h + 1) * DH)
            qh = qb[rows, cols]
            o_acc = jnp.zeros((S_PER, DH), F32)
            l_acc = jnp.zeros((S_PER, 1), F32)
            for kv in range(2):
                krows = slice(kv * (SEQ // 2), (kv + 1) * (SEQ // 2))
                s = lax.dot_general(
                    qh, kb[krows, cols], (((1,), (1,)), ((), ())),
                    preferred_element_type=F32) * SCALE
                p = jnp.exp(s)
                l_acc = l_acc + jnp.sum(p, axis=-1, keepdims=True)
                o_acc = o_acc + jnp.dot(p.astype(BF16), vb[krows, cols],
                                        preferred_element_type=F32)
            attnbuf[:, cols] = (o_acc / l_acc).astype(BF16)
        return jnp.dot(attnbuf[...], wo, preferred_element_type=F32)

    def part_copy(src, slot, target):
        return pltpu.make_async_remote_copy(
            src_ref=src,
            dst_ref=rsbuf.at[slot],
            send_sem=rs_send.at[slot],
            recv_sem=rs_recv.at[slot],
            device_id=(target,),
            device_id_type=pl.DeviceIdType.MESH,
        )

    pacc[...] = pchunk((i + 2) % N_DEV).astype(BF16)
    rdma_d = part_copy(pacc, 0, right)
    rdma_d.start()
    p_next = pchunk((i + 1) % N_DEV)
    rdma_d.wait_recv()
    rdma_d.wait_send()
    pacc[...] = (rsbuf[0].astype(F32) + p_next).astype(BF16)
    rdma_c = part_copy(pacc, 1, right)
    rdma_c.start()
    pacc2[...] = pchunk((i + N_DEV - 1) % N_DEV).astype(BF16)
    rdma_rs_l = part_copy(pacc2, 2, left)
    rdma_rs_l.start()
    own = pchunk(i)
    rdma_c.wait_recv()
    rdma_rs_l.wait_recv()
    out_ref[...] = own + rsbuf[1].astype(F32) + rsbuf[2].astype(F32)
    rdma_c.wait_send()
    rdma_rs_l.wait_send()


def kernel(x, Wq, Wo, Wk, Wv):
    x2 = x.reshape(S_PER, D)

    out = pl.pallas_call(
        _body,
        out_shape=jax.ShapeDtypeStruct((S_PER, D), F32),
        in_specs=[pl.BlockSpec(memory_space=pltpu.VMEM)] * 5,
        out_specs=pl.BlockSpec(memory_space=pltpu.VMEM),
        scratch_shapes=[
            pltpu.VMEM((SEQ, D), BF16),
            pltpu.VMEM((SEQ, D), BF16),
            pltpu.VMEM((SEQ, D), BF16),
            pltpu.VMEM((SEQ, D), BF16),
            pltpu.VMEM((S_PER, D), BF16),
            pltpu.VMEM((S_PER, D), BF16),
            pltpu.VMEM((S_PER, D), BF16),
            pltpu.VMEM((N_DEV - 1, S_PER, D), BF16),
            pltpu.SemaphoreType.DMA((N_DEV - 1,)),
            pltpu.SemaphoreType.DMA((N_DEV - 1,)),
            pltpu.SemaphoreType.DMA((N_DEV - 1,)),
            pltpu.SemaphoreType.DMA((N_DEV - 1,)),
        ],
        compiler_params=pltpu.CompilerParams(
            collective_id=0, vmem_limit_bytes=60 * 1024 * 1024
        ),
    )(x2, Wq, Wo, Wk, Wv)
    return out.reshape(1, S_PER, D)


# device time: 94092 ns/iter; 1.0352x vs baseline; 1.0352x over previous
import jax
import jax.numpy as jnp
from jax import lax
from jax.experimental import pallas as pl
from jax.experimental.pallas import tpu as pltpu

N_DEV = 4
S_PER = 512
SEQ = N_DEV * S_PER
D = 1024
H = 8
DH = 128
SCALE = 0.08838834764831843
BF16 = jnp.bfloat16
F32 = jnp.float32


def _body(x_ref, wq_ref, wo_ref, wk_ref, wv_ref, out_ref,
          xc, qb, kb, vb, attnbuf, pacc, pacc2, rsbuf,
          ag_send, ag_recv, rs_send, rs_recv):
    i = lax.axis_index("i")
    left = (i + N_DEV - 1) % N_DEV
    right = (i + 1) % N_DEV

    barrier = pltpu.get_barrier_semaphore()
    for nbr in (left, right):
        pl.semaphore_signal(barrier, inc=1, device_id=(nbr,),
                            device_id_type=pl.DeviceIdType.MESH)
    pl.semaphore_wait(barrier, 2)

    xc[pl.ds(i * S_PER, S_PER), :] = x_ref[...].astype(BF16)
    wq = wq_ref[...].astype(BF16)
    wk = wk_ref[...].astype(BF16)
    wv = wv_ref[...].astype(BF16)

    def project(c):
        rows = pl.ds(c * S_PER, S_PER)
        xv = xc[rows, :]
        qb[rows, :] = jnp.dot(xv, wq, preferred_element_type=F32).astype(BF16)
        kb[rows, :] = jnp.dot(xv, wk, preferred_element_type=F32).astype(BF16)
        vb[rows, :] = jnp.dot(xv, wv, preferred_element_type=F32).astype(BF16)

    def chunk_copy(c, sem_idx, target):
        return pltpu.make_async_remote_copy(
            src_ref=xc.at[pl.ds(c * S_PER, S_PER)],
            dst_ref=xc.at[pl.ds(c * S_PER, S_PER)],
            send_sem=ag_send.at[sem_idx],
            recv_sem=ag_recv.at[sem_idx],
            device_id=(target,),
            device_id_type=pl.DeviceIdType.MESH,
        )

    rdma_r = chunk_copy(i, 0, right)
    rdma_l = chunk_copy(i, 1, left)
    rdma_r.start()
    rdma_l.start()
    project(i)
    rdma_r.wait_recv()
    rdma_f = chunk_copy((i + N_DEV - 1) % N_DEV, 2, right)
    rdma_f.start()
    rdma_l.wait_recv()
    project((i + N_DEV - 1) % N_DEV)
    project((i + 1) % N_DEV)
    rdma_f.wait_recv()
    project((i + 2) % N_DEV)
    rdma_r.wait_send()
    rdma_l.wait_send()
    rdma_f.wait_send()

    wo = wo_ref[...].astype(BF16)

    def pchunk(c):
        rows = pl.ds(c * S_PER, S_PER)

        def head_body(hh, carry):
            for u in range(4):
                cols = pl.ds((4 * hh + u) * DH, DH)
                qh = qb[rows, cols]
                s = lax.dot_general(
                    qh, kb[:, cols], (((1,), (1,)), ((), ())),
                    preferred_element_type=F32) * SCALE
                p = jnp.exp(s)
                l = jnp.sum(p, axis=-1, keepdims=True)
                o = jnp.dot(p.astype(BF16), vb[:, cols],
                            preferred_element_type=F32) / l
                attnbuf[:, cols] = o.astype(BF16)
            return carry

        lax.fori_loop(0, H // 4, head_body, 0)
        return jnp.dot(attnbuf[...], wo, preferred_element_type=F32)

    def part_copy(src, slot, target):
        return pltpu.make_async_remote_copy(
            src_ref=src,
            dst_ref=rsbuf.at[slot],
            send_sem=rs_send.at[slot],
            recv_sem=rs_recv.at[slot],
            device_id=(target,),
            device_id_type=pl.DeviceIdType.MESH,
        )

    pacc[...] = pchunk((i + 2) % N_DEV).astype(BF16)
    rdma_d = part_copy(pacc, 0, right)
    rdma_d.start()
    p_next = pchunk((i + 1) % N_DEV)
    rdma_d.wait_recv()
    rdma_d.wait_send()
    pacc[...] = (rsbuf[0].astype(F32) + p_next).astype(BF16)
    rdma_c = part_copy(pacc, 1, right)
    rdma_c.start()
    pacc2[...] = pchunk((i + N_DEV - 1) % N_DEV).astype(BF16)
    rdma_rs_l = part_copy(pacc2, 2, left)
    rdma_rs_l.start()
    own = pchunk(i)
    rdma_c.wait_recv()
    rdma_rs_l.wait_recv()
    out_ref[...] = own + rsbuf[1].astype(F32) + rsbuf[2].astype(F32)
    rdma_c.wait_send()
    rdma_rs_l.wait_send()


def kernel(x, Wq, Wo, Wk, Wv):
    x2 = x.reshape(S_PER, D)

    out = pl.pallas_call(
        _body,
        out_shape=jax.ShapeDtypeStruct((S_PER, D), F32),
        in_specs=[pl.BlockSpec(memory_space=pltpu.VMEM)] * 5,
        out_specs=pl.BlockSpec(memory_space=pltpu.VMEM),
        scratch_shapes=[
            pltpu.VMEM((SEQ, D), BF16),
            pltpu.VMEM((SEQ, D), BF16),
            pltpu.VMEM((SEQ, D), BF16),
            pltpu.VMEM((SEQ, D), BF16),
            pltpu.VMEM((S_PER, D), BF16),
            pltpu.VMEM((S_PER, D), BF16),
            pltpu.VMEM((S_PER, D), BF16),
            pltpu.VMEM((N_DEV - 1, S_PER, D), BF16),
            pltpu.SemaphoreType.DMA((N_DEV - 1,)),
            pltpu.SemaphoreType.DMA((N_DEV - 1,)),
            pltpu.SemaphoreType.DMA((N_DEV - 1,)),
            pltpu.SemaphoreType.DMA((N_DEV - 1,)),
        ],
        compiler_params=pltpu.CompilerParams(
            collective_id=0, vmem_limit_bytes=60 * 1024 * 1024
        ),
    )(x2, Wq, Wo, Wk, Wv)
    return out.reshape(1, S_PER, D)
